# baseline (speedup 1.0000x reference)
import functools

import jax
import jax.numpy as jnp
from jax import lax
from jax.experimental import pallas as pl
from jax.experimental.pallas import tpu as pltpu
from jax.experimental.pallas import tpu_sc as plsc

N = 10000
E = 320000
D = 128
H = 128
C = 40

NC = 2
NS = 16
NW = NC * NS
EPW = E // NW
CH = 80
NCHUNK = EPW // CH
NP = 10240
ROWS_PT = NP // NS


def _sc_agg_body(eidx_hbm, x_hbm, zf_hbm, part_hbm,
                 acc, ib, rows_v, gsem0, gsem1, hsem0, hsem1, isem0, isem1):
    cid = lax.axis_index("c")
    sid = lax.axis_index("s")
    wid = cid * NS + sid

    pltpu.sync_copy(zf_hbm.at[pl.ds(sid * ROWS_PT, ROWS_PT)],
                    acc.at[pl.ds(sid * ROWS_PT, ROWS_PT)])
    plsc.subcore_barrier()

    gsem = (gsem0, gsem1)
    hsem = (hsem0, hsem1)
    isem = (isem0, isem1)
    HALF = CH // 2

    def gather(b):
        pltpu.async_copy(x_hbm.at[ib.at[b, 0, pl.ds(0, HALF)]],
                         rows_v.at[b, pl.ds(0, HALF)], gsem[b])
        pltpu.async_copy(x_hbm.at[ib.at[b, 0, pl.ds(HALF, HALF)]],
                         rows_v.at[b, pl.ds(HALF, HALF)], hsem[b])

    def gather_wait(b):
        pltpu.make_async_copy(x_hbm.at[ib.at[b, 0, pl.ds(0, HALF)]],
                              rows_v.at[b, pl.ds(0, HALF)], gsem[b]).wait()
        pltpu.make_async_copy(x_hbm.at[ib.at[b, 0, pl.ds(HALF, HALF)]],
                              rows_v.at[b, pl.ds(HALF, HALF)], hsem[b]).wait()

    pltpu.sync_copy(eidx_hbm.at[wid, 0], ib.at[0])
    gather(0)
    pltpu.async_copy(eidx_hbm.at[wid, 1], ib.at[1], isem[1])

    def step(j, cur, nxt):
        @pl.when(j + 1 < NCHUNK)
        def _():
            pltpu.make_async_copy(eidx_hbm.at[wid, j + 1], ib.at[nxt],
                                  isem[nxt]).wait()
            gather(nxt)
        gather_wait(cur)
        pltpu.sync_copy(rows_v.at[cur], acc.at[ib.at[cur, 1]], add=True)

        @pl.when(j + 2 < NCHUNK)
        def _():
            pltpu.async_copy(eidx_hbm.at[wid, j + 2], ib.at[cur], isem[cur])

    def chunk(j, _):
        @pl.when(j % 2 == 0)
        def _():
            step(j, 0, 1)

        @pl.when(j % 2 == 1)
        def _():
            step(j, 1, 0)
        return 0

    lax.fori_loop(0, NCHUNK, chunk, 0)
    plsc.subcore_barrier()

    pltpu.sync_copy(acc.at[pl.ds(sid * ROWS_PT, ROWS_PT)],
                    part_hbm.at[cid, pl.ds(sid * ROWS_PT, ROWS_PT)])


def _make_sc_agg(w=D):
    mesh = plsc.VectorSubcoreMesh(core_axis_name="c", subcore_axis_name="s")
    return pl.kernel(
        _sc_agg_body,
        out_type=jax.ShapeDtypeStruct((NC, NP, w), jnp.float32),
        mesh=mesh,
        scratch_types=[
            pltpu.VMEM_SHARED((NP, w), jnp.float32),
            pltpu.VMEM((2, 2, CH), jnp.int32),
            pltpu.VMEM((2, CH, w), jnp.float32),
            pltpu.SemaphoreType.DMA,
            pltpu.SemaphoreType.DMA,
            pltpu.SemaphoreType.DMA,
            pltpu.SemaphoreType.DMA,
            pltpu.SemaphoreType.DMA,
            pltpu.SemaphoreType.DMA,
        ],
    )


def _sc_cnt_body(dst_hbm, zf_hbm, ones_hbm, pcnt_hbm,
                 acc_cnt, dst_v, ones_v):
    cid = lax.axis_index("c")
    sid = lax.axis_index("s")
    wid = cid * NS + sid

    pltpu.sync_copy(zf_hbm.at[pl.ds(sid * ROWS_PT, ROWS_PT)],
                    acc_cnt.at[pl.ds(sid * ROWS_PT, ROWS_PT)])
    pltpu.sync_copy(ones_hbm, ones_v)
    pltpu.sync_copy(dst_hbm.at[wid], dst_v)
    plsc.subcore_barrier()

    def chunk(j, _):
        pltpu.sync_copy(ones_v, acc_cnt.at[dst_v.at[j]], add=True)
        return 0

    lax.fori_loop(0, NCHUNK, chunk, 0)
    plsc.subcore_barrier()

    pltpu.sync_copy(acc_cnt.at[pl.ds(sid * ROWS_PT, ROWS_PT)],
                    pcnt_hbm.at[cid, pl.ds(sid * ROWS_PT, ROWS_PT)])


def _make_sc_cnt():
    mesh = plsc.VectorSubcoreMesh(core_axis_name="c", subcore_axis_name="s")
    return pl.kernel(
        _sc_cnt_body,
        out_type=jax.ShapeDtypeStruct((NC, NP, D), jnp.float32),
        mesh=mesh,
        scratch_types=[
            pltpu.VMEM_SHARED((NP, D), jnp.float32),
            pltpu.VMEM((NCHUNK, CH), jnp.int32),
            pltpu.VMEM((CH, D), jnp.float32),
        ],
    )


def _tc_pre_body(x_ref, wr_ref, b_ref, out_ref):
    out_ref[...] = jnp.dot(x_ref[...], wr_ref[...],
                           precision=lax.Precision.HIGHEST,
                           preferred_element_type=jnp.float32) + b_ref[...]


def _tc_pre(x, wr, b):
    bn = 1000
    fout = wr.shape[1]
    return pl.pallas_call(
        _tc_pre_body,
        grid=(N // bn,),
        in_specs=[
            pl.BlockSpec((bn, D), lambda i: (i, 0)),
            pl.BlockSpec((D, fout), lambda i: (0, 0)),
            pl.BlockSpec((1, fout), lambda i: (0, 0)),
        ],
        out_specs=pl.BlockSpec((bn, fout), lambda i: (i, 0)),
        out_shape=jax.ShapeDtypeStruct((N, fout), jnp.float32),
    )(x, wr, b.reshape(1, fout))


def _tc_post_body(relu, part_ref, pcnt_ref, v_ref, wl_ref, out_ref):
    cnt = pcnt_ref[0, :, 0] + pcnt_ref[1, :, 0]
    inv = 1.0 / jnp.maximum(cnt, 1.0)
    u = jnp.dot(part_ref[0] + part_ref[1], wl_ref[...],
                precision=lax.Precision.HIGHEST,
                preferred_element_type=jnp.float32)
    acc = u * inv[:, None] + v_ref[...]
    out_ref[...] = jnp.maximum(acc, 0.0) if relu else acc


def _tc_post(part, pcnt, v, wl, relu):
    bn = 1000
    fout = wl.shape[1]
    return pl.pallas_call(
        functools.partial(_tc_post_body, relu),
        grid=(N // bn,),
        in_specs=[
            pl.BlockSpec((NC, bn, D), lambda i: (0, i, 0)),
            pl.BlockSpec((NC, bn, 8), lambda i: (0, i, 0)),
            pl.BlockSpec((bn, fout), lambda i: (i, 0)),
            pl.BlockSpec((D, fout), lambda i: (0, 0)),
        ],
        out_specs=pl.BlockSpec((bn, fout), lambda i: (i, 0)),
        out_shape=jax.ShapeDtypeStruct((N, fout), jnp.float32),
    )(part, pcnt, v, wl)


_sc_agg = _make_sc_agg()
_sc_agg48 = _make_sc_agg(48)
_sc_cnt = _make_sc_cnt()


def kernel(x, edge_index, Wl1, Wr1, b1, Wl2, Wr2, b2, Wl3, Wr3, b3):
    src = edge_index[0].reshape(NW, NCHUNK, CH)
    dst = edge_index[1].reshape(NW, NCHUNK, CH)
    eidx = jnp.stack([src, dst], axis=2)
    zf = jnp.zeros((NP, D), jnp.float32)
    ones = jnp.ones((CH, D), jnp.float32)

    pcnt = _sc_cnt(dst, zf, ones)[:, :, :8]
    part1 = _sc_agg(eidx, x, zf)
    v1 = _tc_pre(x, Wr1, b1)
    h1 = _tc_post(part1, pcnt, v1, Wl1, relu=True)
    part2 = _sc_agg(eidx, h1, zf)
    v2 = _tc_pre(h1, Wr2, b2)
    h2 = _tc_post(part2, pcnt, v2, Wl2, relu=True)
    part3 = _sc_agg(eidx, h2, zf)
    v3 = _tc_pre(h2, Wr3, b3)
    out = _tc_post(part3, pcnt, v3, Wl3, relu=False)
    return out


# speedup vs baseline: 1.0594x; 1.0594x over previous
import functools

import jax
import jax.numpy as jnp
from jax import lax
from jax.experimental import pallas as pl
from jax.experimental.pallas import tpu as pltpu
from jax.experimental.pallas import tpu_sc as plsc

N = 10000
E = 320000
D = 128
H = 128
C = 40

NC = 2
NS = 16
NW = NC * NS
EPW = E // NW
CH = 100
NCHUNK = EPW // CH
NP = 10240
ROWS_PT = NP // NS


def _sc_agg_body(eidx_hbm, x_hbm, zf_hbm, part_hbm,
                 acc, ib, rows_v, gsem0, gsem1, isem0, isem1):
    cid = lax.axis_index("c")
    sid = lax.axis_index("s")
    wid = cid * NS + sid

    pltpu.sync_copy(zf_hbm.at[pl.ds(sid * ROWS_PT, ROWS_PT)],
                    acc.at[pl.ds(sid * ROWS_PT, ROWS_PT)])
    plsc.subcore_barrier()

    gsem = (gsem0, gsem1)
    isem = (isem0, isem1)

    pltpu.sync_copy(eidx_hbm.at[wid, 0], ib.at[0])
    pltpu.async_copy(x_hbm.at[ib.at[0, 0]], rows_v.at[0], gsem[0])
    pltpu.async_copy(eidx_hbm.at[wid, 1], ib.at[1], isem[1])

    def step(j, cur, nxt):
        @pl.when(j + 1 < NCHUNK)
        def _():
            pltpu.make_async_copy(eidx_hbm.at[wid, j + 1], ib.at[nxt],
                                  isem[nxt]).wait()
            pltpu.async_copy(x_hbm.at[ib.at[nxt, 0]], rows_v.at[nxt],
                             gsem[nxt])
        pltpu.make_async_copy(x_hbm.at[ib.at[cur, 0]], rows_v.at[cur],
                              gsem[cur]).wait()
        pltpu.sync_copy(rows_v.at[cur], acc.at[ib.at[cur, 1]], add=True)

        @pl.when(j + 2 < NCHUNK)
        def _():
            pltpu.async_copy(eidx_hbm.at[wid, j + 2], ib.at[cur], isem[cur])

    def chunk(j, _):
        @pl.when(j % 2 == 0)
        def _():
            step(j, 0, 1)

        @pl.when(j % 2 == 1)
        def _():
            step(j, 1, 0)
        return 0

    lax.fori_loop(0, NCHUNK, chunk, 0)
    plsc.subcore_barrier()

    pltpu.sync_copy(acc.at[pl.ds(sid * ROWS_PT, ROWS_PT)],
                    part_hbm.at[cid, pl.ds(sid * ROWS_PT, ROWS_PT)])


def _make_sc_agg(w=D):
    mesh = plsc.VectorSubcoreMesh(core_axis_name="c", subcore_axis_name="s")
    return pl.kernel(
        _sc_agg_body,
        out_type=jax.ShapeDtypeStruct((NC, NP, w), jnp.float32),
        mesh=mesh,
        scratch_types=[
            pltpu.VMEM_SHARED((NP, w), jnp.float32),
            pltpu.VMEM((2, 2, CH), jnp.int32),
            pltpu.VMEM((2, CH, w), jnp.float32),
            pltpu.SemaphoreType.DMA,
            pltpu.SemaphoreType.DMA,
            pltpu.SemaphoreType.DMA,
            pltpu.SemaphoreType.DMA,
        ],
    )


def _sc_cnt_body(dst_hbm, zf_hbm, ones_hbm, pcnt_hbm,
                 acc_cnt, dst_v, ones_v):
    cid = lax.axis_index("c")
    sid = lax.axis_index("s")
    wid = cid * NS + sid

    pltpu.sync_copy(zf_hbm.at[pl.ds(sid * ROWS_PT, ROWS_PT)],
                    acc_cnt.at[pl.ds(sid * ROWS_PT, ROWS_PT)])
    pltpu.sync_copy(ones_hbm, ones_v)
    pltpu.sync_copy(dst_hbm.at[wid], dst_v)
    plsc.subcore_barrier()

    def chunk(j, _):
        pltpu.sync_copy(ones_v, acc_cnt.at[dst_v.at[j]], add=True)
        return 0

    lax.fori_loop(0, NCHUNK, chunk, 0)
    plsc.subcore_barrier()

    pltpu.sync_copy(acc_cnt.at[pl.ds(sid * ROWS_PT, ROWS_PT)],
                    pcnt_hbm.at[cid, pl.ds(sid * ROWS_PT, ROWS_PT)])


def _make_sc_cnt():
    mesh = plsc.VectorSubcoreMesh(core_axis_name="c", subcore_axis_name="s")
    return pl.kernel(
        _sc_cnt_body,
        out_type=jax.ShapeDtypeStruct((NC, NP, D), jnp.float32),
        mesh=mesh,
        scratch_types=[
            pltpu.VMEM_SHARED((NP, D), jnp.float32),
            pltpu.VMEM((NCHUNK, CH), jnp.int32),
            pltpu.VMEM((CH, D), jnp.float32),
        ],
    )


def _tc_pre_body(x_ref, wr_ref, b_ref, out_ref):
    out_ref[...] = jnp.dot(x_ref[...], wr_ref[...],
                           precision=lax.Precision.HIGHEST,
                           preferred_element_type=jnp.float32) + b_ref[...]


def _tc_pre(x, wr, b):
    bn = 1000
    fout = wr.shape[1]
    return pl.pallas_call(
        _tc_pre_body,
        grid=(N // bn,),
        in_specs=[
            pl.BlockSpec((bn, D), lambda i: (i, 0)),
            pl.BlockSpec((D, fout), lambda i: (0, 0)),
            pl.BlockSpec((1, fout), lambda i: (0, 0)),
        ],
        out_specs=pl.BlockSpec((bn, fout), lambda i: (i, 0)),
        out_shape=jax.ShapeDtypeStruct((N, fout), jnp.float32),
    )(x, wr, b.reshape(1, fout))


def _tc_post_body(relu, part_ref, pcnt_ref, v_ref, wl_ref, out_ref):
    cnt = pcnt_ref[0, :, 0] + pcnt_ref[1, :, 0]
    inv = 1.0 / jnp.maximum(cnt, 1.0)
    u = jnp.dot(part_ref[0] + part_ref[1], wl_ref[...],
                precision=lax.Precision.HIGHEST,
                preferred_element_type=jnp.float32)
    acc = u * inv[:, None] + v_ref[...]
    out_ref[...] = jnp.maximum(acc, 0.0) if relu else acc


def _tc_post(part, pcnt, v, wl, relu):
    bn = 1000
    fout = wl.shape[1]
    return pl.pallas_call(
        functools.partial(_tc_post_body, relu),
        grid=(N // bn,),
        in_specs=[
            pl.BlockSpec((NC, bn, D), lambda i: (0, i, 0)),
            pl.BlockSpec((NC, bn, 8), lambda i: (0, i, 0)),
            pl.BlockSpec((bn, fout), lambda i: (i, 0)),
            pl.BlockSpec((D, fout), lambda i: (0, 0)),
        ],
        out_specs=pl.BlockSpec((bn, fout), lambda i: (i, 0)),
        out_shape=jax.ShapeDtypeStruct((N, fout), jnp.float32),
    )(part, pcnt, v, wl)


_sc_agg = _make_sc_agg()
_sc_agg48 = _make_sc_agg(48)
_sc_cnt = _make_sc_cnt()


def kernel(x, edge_index, Wl1, Wr1, b1, Wl2, Wr2, b2, Wl3, Wr3, b3):
    src = edge_index[0].reshape(NW, NCHUNK, CH)
    dst = edge_index[1].reshape(NW, NCHUNK, CH)
    eidx = jnp.stack([src, dst], axis=2)
    zf = jnp.zeros((NP, D), jnp.float32)
    ones = jnp.ones((CH, D), jnp.float32)

    pcnt = _sc_cnt(dst, zf, ones)[:, :, :8]
    part1 = _sc_agg(eidx, x, zf)
    v1 = _tc_pre(x, Wr1, b1)
    h1 = _tc_post(part1, pcnt, v1, Wl1, relu=True)
    part2 = _sc_agg(eidx, h1, zf)
    v2 = _tc_pre(h1, Wr2, b2)
    h2 = _tc_post(part2, pcnt, v2, Wl2, relu=True)
    part3 = _sc_agg(eidx, h2, zf)
    v3 = _tc_pre(h2, Wr3, b3)
    out = _tc_post(part3, pcnt, v3, Wl3, relu=False)
    return out


# speedup vs baseline: 1.0849x; 1.0241x over previous
import functools

import jax
import jax.numpy as jnp
from jax import lax
from jax.experimental import pallas as pl
from jax.experimental.pallas import tpu as pltpu
from jax.experimental.pallas import tpu_sc as plsc

N = 10000
E = 320000
D = 128
H = 128
C = 40

NC = 2
NS = 16
NW = NC * NS
EPW = E // NW
CH = 100
NCHUNK = EPW // CH
NP = 10240
ROWS_PT = NP // NS


def _sc_agg_body(eidx_hbm, x_hbm, zf_hbm, part_hbm,
                 acc, ib, rows_v, gsem0, gsem1, isem0, isem1):
    cid = lax.axis_index("c")
    sid = lax.axis_index("s")
    wid = cid * NS + sid

    pltpu.sync_copy(zf_hbm.at[pl.ds(sid * ROWS_PT, ROWS_PT)],
                    acc.at[pl.ds(sid * ROWS_PT, ROWS_PT)])
    plsc.subcore_barrier()

    gsem = (gsem0, gsem1)
    isem = (isem0, isem1)

    pltpu.sync_copy(eidx_hbm.at[wid, 0], ib.at[0])
    pltpu.async_copy(x_hbm.at[ib.at[0, 0]], rows_v.at[0], gsem[0])
    pltpu.async_copy(eidx_hbm.at[wid, 1], ib.at[1], isem[1])

    def step(j, cur, nxt):
        @pl.when(j + 1 < NCHUNK)
        def _():
            pltpu.make_async_copy(eidx_hbm.at[wid, j + 1], ib.at[nxt],
                                  isem[nxt]).wait()
            pltpu.async_copy(x_hbm.at[ib.at[nxt, 0]], rows_v.at[nxt],
                             gsem[nxt])
        pltpu.make_async_copy(x_hbm.at[ib.at[cur, 0]], rows_v.at[cur],
                              gsem[cur]).wait()
        pltpu.sync_copy(rows_v.at[cur], acc.at[ib.at[cur, 1]], add=True)

        @pl.when(j + 2 < NCHUNK)
        def _():
            pltpu.async_copy(eidx_hbm.at[wid, j + 2], ib.at[cur], isem[cur])

    def chunk(j, _):
        @pl.when(j % 2 == 0)
        def _():
            step(j, 0, 1)

        @pl.when(j % 2 == 1)
        def _():
            step(j, 1, 0)
        return 0

    lax.fori_loop(0, NCHUNK, chunk, 0)
    plsc.subcore_barrier()

    pltpu.sync_copy(acc.at[pl.ds(sid * ROWS_PT, ROWS_PT)],
                    part_hbm.at[cid, pl.ds(sid * ROWS_PT, ROWS_PT)])


def _make_sc_agg(w=D):
    mesh = plsc.VectorSubcoreMesh(core_axis_name="c", subcore_axis_name="s")
    return pl.kernel(
        _sc_agg_body,
        out_type=jax.ShapeDtypeStruct((NC, NP, w), jnp.float32),
        mesh=mesh,
        scratch_types=[
            pltpu.VMEM_SHARED((NP, w), jnp.float32),
            pltpu.VMEM((2, 2, CH), jnp.int32),
            pltpu.VMEM((2, CH, w), jnp.float32),
            pltpu.SemaphoreType.DMA,
            pltpu.SemaphoreType.DMA,
            pltpu.SemaphoreType.DMA,
            pltpu.SemaphoreType.DMA,
        ],
    )


def _sc_cnt_body(dst_hbm, zf_hbm, ones_hbm, pcnt_hbm,
                 acc_cnt, dst_v, ones_v, csem0, csem1):
    cid = lax.axis_index("c")
    sid = lax.axis_index("s")
    wid = cid * NS + sid

    pltpu.sync_copy(zf_hbm.at[pl.ds(sid * ROWS_PT, ROWS_PT)],
                    acc_cnt.at[pl.ds(sid * ROWS_PT, ROWS_PT)])
    pltpu.sync_copy(ones_hbm, ones_v)
    pltpu.sync_copy(dst_hbm.at[wid], dst_v)
    plsc.subcore_barrier()

    csem = (csem0, csem1)
    pltpu.async_copy(ones_v, acc_cnt.at[dst_v.at[0]], csem[0], add=True)

    def cstep(j, cur, nxt):
        @pl.when(j + 1 < NCHUNK)
        def _():
            pltpu.async_copy(ones_v, acc_cnt.at[dst_v.at[j + 1]], csem[nxt],
                             add=True)
        pltpu.make_async_copy(ones_v, acc_cnt.at[dst_v.at[j]],
                              csem[cur]).wait()

    def chunk(j, _):
        @pl.when(j % 2 == 0)
        def _():
            cstep(j, 0, 1)

        @pl.when(j % 2 == 1)
        def _():
            cstep(j, 1, 0)
        return 0

    lax.fori_loop(0, NCHUNK, chunk, 0)
    plsc.subcore_barrier()

    pltpu.sync_copy(acc_cnt.at[pl.ds(sid * ROWS_PT, ROWS_PT)],
                    pcnt_hbm.at[cid, pl.ds(sid * ROWS_PT, ROWS_PT)])


def _make_sc_cnt():
    mesh = plsc.VectorSubcoreMesh(core_axis_name="c", subcore_axis_name="s")
    return pl.kernel(
        _sc_cnt_body,
        out_type=jax.ShapeDtypeStruct((NC, NP, D), jnp.float32),
        mesh=mesh,
        scratch_types=[
            pltpu.VMEM_SHARED((NP, D), jnp.float32),
            pltpu.VMEM((NCHUNK, CH), jnp.int32),
            pltpu.VMEM((CH, D), jnp.float32),
            pltpu.SemaphoreType.DMA,
            pltpu.SemaphoreType.DMA,
        ],
    )


def _tc_pre_body(x_ref, wr_ref, b_ref, out_ref):
    out_ref[...] = jnp.dot(x_ref[...], wr_ref[...],
                           precision=lax.Precision.HIGHEST,
                           preferred_element_type=jnp.float32) + b_ref[...]


def _tc_pre(x, wr, b):
    bn = 2000
    fout = wr.shape[1]
    return pl.pallas_call(
        _tc_pre_body,
        grid=(N // bn,),
        in_specs=[
            pl.BlockSpec((bn, D), lambda i: (i, 0)),
            pl.BlockSpec((D, fout), lambda i: (0, 0)),
            pl.BlockSpec((1, fout), lambda i: (0, 0)),
        ],
        out_specs=pl.BlockSpec((bn, fout), lambda i: (i, 0)),
        out_shape=jax.ShapeDtypeStruct((N, fout), jnp.float32),
    )(x, wr, b.reshape(1, fout))


def _tc_post_body(relu, part_ref, pcnt_ref, v_ref, wl_ref, out_ref):
    cnt = pcnt_ref[0, :, 0] + pcnt_ref[1, :, 0]
    inv = 1.0 / jnp.maximum(cnt, 1.0)
    u = jnp.dot(part_ref[0] + part_ref[1], wl_ref[...],
                precision=lax.Precision.HIGHEST,
                preferred_element_type=jnp.float32)
    acc = u * inv[:, None] + v_ref[...]
    out_ref[...] = jnp.maximum(acc, 0.0) if relu else acc


def _tc_post(part, pcnt, v, wl, relu):
    bn = 2000
    fout = wl.shape[1]
    return pl.pallas_call(
        functools.partial(_tc_post_body, relu),
        grid=(N // bn,),
        in_specs=[
            pl.BlockSpec((NC, bn, D), lambda i: (0, i, 0)),
            pl.BlockSpec((NC, bn, 8), lambda i: (0, i, 0)),
            pl.BlockSpec((bn, fout), lambda i: (i, 0)),
            pl.BlockSpec((D, fout), lambda i: (0, 0)),
        ],
        out_specs=pl.BlockSpec((bn, fout), lambda i: (i, 0)),
        out_shape=jax.ShapeDtypeStruct((N, fout), jnp.float32),
    )(part, pcnt, v, wl)


_sc_agg = _make_sc_agg()
_sc_agg48 = _make_sc_agg(48)
_sc_cnt = _make_sc_cnt()


def kernel(x, edge_index, Wl1, Wr1, b1, Wl2, Wr2, b2, Wl3, Wr3, b3):
    src = edge_index[0].reshape(NW, NCHUNK, CH)
    dst = edge_index[1].reshape(NW, NCHUNK, CH)
    eidx = jnp.stack([src, dst], axis=2)
    zf = jnp.zeros((NP, D), jnp.float32)
    ones = jnp.ones((CH, D), jnp.float32)

    pcnt = _sc_cnt(dst, zf, ones)[:, :, :8]
    part1 = _sc_agg(eidx, x, zf)
    v1 = _tc_pre(x, Wr1, b1)
    h1 = _tc_post(part1, pcnt, v1, Wl1, relu=True)
    part2 = _sc_agg(eidx, h1, zf)
    v2 = _tc_pre(h1, Wr2, b2)
    h2 = _tc_post(part2, pcnt, v2, Wl2, relu=True)
    part3 = _sc_agg(eidx, h2, zf)
    v3 = _tc_pre(h2, Wr3, b3)
    out = _tc_post(part3, pcnt, v3, Wl3, relu=False)
    return out
